# Initial kernel scaffold; baseline (speedup 1.0000x reference)
import jax
import jax.numpy as jnp
from jax.experimental import pallas as pl


def kernel(x, edge_index, W, bn_gamma, bn_beta):
    raise NotImplementedError("write your pallas kernel here")


# speedup vs baseline: 9.5665x; 9.5665x over previous
import jax
import jax.numpy as jnp
from jax import lax
from jax.experimental import pallas as pl
from jax.experimental.pallas import tpu as pltpu
from jax.experimental.pallas import tpu_sc as plsc

N = 10000
E = 160000
D = 256
DQ = 64
NQ = D // DQ
NPAD = 10240
NC = 2
NS = 16
SEG = NPAD // NS

K1K = 40
K1CH = (E // (NC * NS)) // K1K

K3K = 40
K3CH = (E // NS) // K3K

BN_EPS_ = 1e-5
RBLK = 400
NBLK = N // RBLK


def _sc_mesh():
    return plsc.VectorSubcoreMesh(
        core_axis_name="c", subcore_axis_name="s", num_cores=NC, num_subcores=NS
    )


def _k1_body(dst_hbm, out_hbm, ones_v, zb_v, dstc_v, deg_sh):
    c = lax.axis_index("c")
    s = lax.axis_index("s")
    w = c * NS + s

    def fill(i, carry):
        zb_v[i] = jnp.zeros((16,), jnp.float32)
        ones_v[i] = jnp.ones((16,), jnp.float32)
        return carry

    lax.fori_loop(0, K1K, fill, 0)

    for r in range(SEG // K1K):
        pltpu.sync_copy(zb_v, deg_sh.at[pl.ds(s * SEG + r * K1K, K1K)])
    plsc.subcore_barrier()

    pltpu.sync_copy(dst_hbm.at[w], dstc_v)

    def body(j, carry):
        pltpu.sync_copy(ones_v, deg_sh.at[dstc_v.at[j]], add=True)
        return carry

    lax.fori_loop(0, K1CH, body, 0)
    plsc.subcore_barrier()
    pltpu.sync_copy(
        deg_sh.at[pl.ds(s * SEG, SEG)], out_hbm.at[c, pl.ds(s * SEG, SEG)]
    )


def _k1_call(dst3):
    kfn = pl.kernel(
        _k1_body,
        out_type=jax.ShapeDtypeStruct((NC, NPAD, 16), jnp.float32),
        mesh=_sc_mesh(),
        compiler_params=pltpu.CompilerParams(use_tc_tiling_on_sc=False),
        scratch_types=[
            pltpu.VMEM((K1K, 16), jnp.float32),
            pltpu.VMEM((K1K, 16), jnp.float32),
            pltpu.VMEM((K1CH, K1K), jnp.int32),
            pltpu.VMEM_SHARED((NPAD, 16), jnp.float32),
        ],
    )
    return kfn(dst3)


def _k2_body(x_ref, w_ref, p_ref, hp_ref):
    h = jnp.dot(x_ref[...], w_ref[0], preferred_element_type=jnp.float32)
    p = p_ref[...]
    deg = p[0] + p[1] + 1.0
    dinv = lax.rsqrt(deg[:, 0:1])
    hp_ref[...] = h * dinv


def _k2_call(x, W, partials):
    return pl.pallas_call(
        _k2_body,
        grid=(NQ, NBLK),
        in_specs=[
            pl.BlockSpec((RBLK, D), lambda q, i: (i, 0)),
            pl.BlockSpec((1, D, DQ), lambda q, i: (q, 0, 0)),
            pl.BlockSpec((NC, RBLK, 16), lambda q, i: (0, i, 0)),
        ],
        out_specs=pl.BlockSpec((RBLK, DQ), lambda q, i: (q * NBLK + i, 0)),
        out_shape=jax.ShapeDtypeStruct((NQ * N, DQ), jnp.float32),
    )(x, W, partials)


def _k3_body(hp_hbm, src_hbm, dst_hbm, out_hbm,
             srcl_v, dstl_v, rows0, rows1, sem0, sem1, acc_sh):
    c = lax.axis_index("c")
    s = lax.axis_index("s")
    epp = E // NS

    pltpu.sync_copy(src_hbm.at[pl.ds(s * epp, epp)], srcl_v)
    pltpu.sync_copy(dst_hbm.at[s], dstl_v)

    rows = (rows0, rows1)
    sems = (sem0, sem1)

    def shift_src(off):
        def adj(j, carry):
            srcl_v[pl.ds(j * 16, 16)] = srcl_v[pl.ds(j * 16, 16)] + off
            return carry

        lax.fori_loop(0, epp // 16, adj, 0)

    def zero_acc():
        def zf(i, carry):
            for q in range(DQ // 16):
                rows0[i, pl.ds(q * 16, 16)] = jnp.zeros((16,), jnp.float32)
            return carry

        lax.fori_loop(0, K3K, zf, 0)
        for r in range(SEG // K3K):
            pltpu.sync_copy(rows0, acc_sh.at[pl.ds(s * SEG + r * K3K, K3K)])

    def start(j, b):
        pltpu.async_copy(
            hp_hbm.at[srcl_v.at[pl.ds(j * K3K, K3K)]], rows[b], sems[b]
        )

    def wait(j, b):
        pltpu.make_async_copy(
            hp_hbm.at[srcl_v.at[pl.ds(j * K3K, K3K)]], rows[b], sems[b]
        ).wait()

    def edge_pass():
        start(0, 0)
        start(1, 1)

        def gbody(g2, carry):
            for b in range(2):
                j = g2 * 2 + b
                wait(j, b)
                pltpu.sync_copy(rows[b], acc_sh.at[dstl_v.at[j]], add=True)

                @pl.when(j + 2 < K3CH)
                def _():
                    start(j + 2, b)
            return carry

        lax.fori_loop(0, K3CH // 2, gbody, 0)

    for p in range(2):
        zero_acc()
        shift_src(c * N if p == 0 else 2 * N)
        plsc.subcore_barrier()
        edge_pass()
        plsc.subcore_barrier()
        pltpu.sync_copy(
            acc_sh.at[pl.ds(s * SEG, SEG)],
            out_hbm.at[2 * p + c, pl.ds(s * SEG, SEG)],
        )


def _k3_call(hp, src, dst3):
    kfn = pl.kernel(
        _k3_body,
        out_type=jax.ShapeDtypeStruct((NQ, NPAD, DQ), jnp.float32),
        mesh=_sc_mesh(),
        compiler_params=pltpu.CompilerParams(use_tc_tiling_on_sc=False),
        scratch_types=[
            pltpu.VMEM((E // NS,), jnp.int32),
            pltpu.VMEM((K3CH, K3K), jnp.int32),
            pltpu.VMEM((K3K, DQ), jnp.float32),
            pltpu.VMEM((K3K, DQ), jnp.float32),
            pltpu.SemaphoreType.DMA,
            pltpu.SemaphoreType.DMA,
            pltpu.VMEM_SHARED((NPAD, DQ), jnp.float32),
        ],
    )
    return kfn(hp, src, dst3)


def _d1_body(acc_ref, hp0_ref, hp1_ref, hp2_ref, hp3_ref, p_ref,
             t_ref, stats_ref):
    i = pl.program_id(0)
    p = p_ref[...]
    deg = p[0] + p[1] + 1.0
    dinv = lax.rsqrt(deg[:, 0:1])
    a = acc_ref[...]
    hps = (hp0_ref, hp1_ref, hp2_ref, hp3_ref)
    tb = jnp.concatenate(
        [a[q] + hps[q][...] for q in range(NQ)], axis=1
    ) * dinv
    t_ref[...] = tb

    @pl.when(i == 0)
    def _():
        stats_ref[...] = jnp.zeros_like(stats_ref)

    stats_ref[...] += jnp.stack([jnp.sum(tb, 0), jnp.sum(tb * tb, 0)], 0)


def _d1_call(acc, hp, partials):
    return pl.pallas_call(
        _d1_body,
        grid=(NBLK,),
        in_specs=[
            pl.BlockSpec((NQ, RBLK, DQ), lambda i: (0, i, 0)),
            pl.BlockSpec((RBLK, DQ), lambda i: (i, 0)),
            pl.BlockSpec((RBLK, DQ), lambda i: (NBLK + i, 0)),
            pl.BlockSpec((RBLK, DQ), lambda i: (2 * NBLK + i, 0)),
            pl.BlockSpec((RBLK, DQ), lambda i: (3 * NBLK + i, 0)),
            pl.BlockSpec((NC, RBLK, 16), lambda i: (0, i, 0)),
        ],
        out_specs=[
            pl.BlockSpec((RBLK, D), lambda i: (i, 0)),
            pl.BlockSpec((2, D), lambda i: (0, 0)),
        ],
        out_shape=[
            jax.ShapeDtypeStruct((N, D), jnp.float32),
            jax.ShapeDtypeStruct((2, D), jnp.float32),
        ],
    )(acc, hp, hp, hp, hp, partials)


def _d2_body(t_ref, stats_ref, g_ref, b_ref, y_ref):
    st = stats_ref[...]
    mean = st[0:1] * (1.0 / N)
    ex2 = st[1:2] * (1.0 / N)
    var = ex2 - mean * mean
    scale = lax.rsqrt(var + BN_EPS_) * g_ref[...]
    y = (t_ref[...] - mean) * scale + b_ref[...]
    y_ref[...] = jnp.maximum(y, 0.0)


def _d2_call(t, stats, gamma, beta):
    return pl.pallas_call(
        _d2_body,
        grid=(NBLK,),
        in_specs=[
            pl.BlockSpec((RBLK, D), lambda i: (i, 0)),
            pl.BlockSpec((2, D), lambda i: (0, 0)),
            pl.BlockSpec((1, D), lambda i: (0, 0)),
            pl.BlockSpec((1, D), lambda i: (0, 0)),
        ],
        out_specs=pl.BlockSpec((RBLK, D), lambda i: (i, 0)),
        out_shape=jax.ShapeDtypeStruct((N, D), jnp.float32),
    )(t, stats, gamma, beta)


def kernel(x, edge_index, W, bn_gamma, bn_beta):
    ei = edge_index.astype(jnp.int32)
    src = ei[0]
    dst = ei[1]
    dst_k1 = dst.reshape(NC * NS, K1CH, K1K)
    dst_k3 = dst.reshape(NS, K3CH, K3K)

    Wq = W.reshape(D, NQ, DQ).transpose(1, 0, 2)

    partials = _k1_call(dst_k1)
    hp = _k2_call(x, Wq, partials)
    acc = _k3_call(hp, src, dst_k3)
    t, stats = _d1_call(acc, hp, partials)
    return _d2_call(t, stats, bn_gamma.reshape(1, D), bn_beta.reshape(1, D))


# speedup vs baseline: 14.2985x; 1.4946x over previous
import jax
import jax.numpy as jnp
from jax import lax
from jax.experimental import pallas as pl
from jax.experimental.pallas import tpu as pltpu
from jax.experimental.pallas import tpu_sc as plsc

N = 10000
E = 160000
D = 256
DQ = 64
NQ = D // DQ
NPAD = 10240
NC = 2
NS = 16
SEG = NPAD // NS

K1K = 40
K1CH = (E // (NC * NS)) // K1K

K3K = 80
K3CH = (E // NS) // K3K

BN_EPS_ = 1e-5
RBLK = 400
NBLK = N // RBLK


def _sc_mesh():
    return plsc.VectorSubcoreMesh(
        core_axis_name="c", subcore_axis_name="s", num_cores=NC, num_subcores=NS
    )


def _k1_body(dst_hbm, out_hbm, ones_v, zb_v, dstc_v, deg_sh):
    c = lax.axis_index("c")
    s = lax.axis_index("s")
    w = c * NS + s

    def fill(i, carry):
        zb_v[i] = jnp.zeros((16,), jnp.float32)
        ones_v[i] = jnp.ones((16,), jnp.float32)
        return carry

    lax.fori_loop(0, K1K, fill, 0)

    for r in range(SEG // K1K):
        pltpu.sync_copy(zb_v, deg_sh.at[pl.ds(s * SEG + r * K1K, K1K)])
    plsc.subcore_barrier()

    pltpu.sync_copy(dst_hbm.at[w], dstc_v)

    def body(j, carry):
        pltpu.sync_copy(ones_v, deg_sh.at[dstc_v.at[j]], add=True)
        return carry

    lax.fori_loop(0, K1CH, body, 0)
    plsc.subcore_barrier()
    pltpu.sync_copy(
        deg_sh.at[pl.ds(s * SEG, SEG)], out_hbm.at[c, pl.ds(s * SEG, SEG)]
    )


def _k1_call(dst3):
    kfn = pl.kernel(
        _k1_body,
        out_type=jax.ShapeDtypeStruct((NC, NPAD, 16), jnp.float32),
        mesh=_sc_mesh(),
        compiler_params=pltpu.CompilerParams(use_tc_tiling_on_sc=False),
        scratch_types=[
            pltpu.VMEM((K1K, 16), jnp.float32),
            pltpu.VMEM((K1K, 16), jnp.float32),
            pltpu.VMEM((K1CH, K1K), jnp.int32),
            pltpu.VMEM_SHARED((NPAD, 16), jnp.float32),
        ],
    )
    return kfn(dst3)


def _k2_body(x_ref, w_ref, p_ref, hp_ref):
    h = jnp.dot(x_ref[...], w_ref[...], preferred_element_type=jnp.float32)
    p = p_ref[...]
    deg = p[0] + p[1] + 1.0
    dinv = lax.rsqrt(deg[:, 0:1])
    for q in range(NQ):
        hp_ref[q] = h[:, q * DQ:(q + 1) * DQ] * dinv


def _k2_call(x, W, partials):
    return pl.pallas_call(
        _k2_body,
        grid=(NBLK,),
        in_specs=[
            pl.BlockSpec((RBLK, D), lambda i: (i, 0)),
            pl.BlockSpec((D, D), lambda i: (0, 0)),
            pl.BlockSpec((NC, RBLK, 16), lambda i: (0, i, 0)),
        ],
        out_specs=pl.BlockSpec((NQ, RBLK, DQ), lambda i: (0, i, 0)),
        out_shape=jax.ShapeDtypeStruct((NQ, N, DQ), jnp.float32),
    )(x, W, partials)


def _k3_body(hp_hbm, src_hbm, dst_hbm, out_hbm,
             srcl_v, dstl_v, rows0, rows1, rows2, rows3,
             gsem0, gsem1, gsem2, gsem3, ssem0, ssem1, ssem2, ssem3,
             acc_sh):
    c = lax.axis_index("c")
    s = lax.axis_index("s")
    epp = E // NS

    rows = (rows0, rows1, rows2, rows3)
    gsems = (gsem0, gsem1, gsem2, gsem3)
    ssems = (ssem0, ssem1, ssem2, ssem3)

    pltpu.sync_copy(src_hbm.at[pl.ds(s * epp, epp)], srcl_v)
    pltpu.sync_copy(dst_hbm.at[s], dstl_v)

    def shift_src(off):
        def adj(j, carry):
            srcl_v[pl.ds(j * 16, 16)] = srcl_v[pl.ds(j * 16, 16)] + off
            return carry

        lax.fori_loop(0, epp // 16, adj, 0)

    def zero_acc():
        def zf(i, carry):
            for q in range(DQ // 16):
                rows0[i, pl.ds(q * 16, 16)] = jnp.zeros((16,), jnp.float32)
            return carry

        lax.fori_loop(0, K3K, zf, 0)
        for r in range(SEG // K3K):
            pltpu.sync_copy(rows0, acc_sh.at[pl.ds(s * SEG + r * K3K, K3K)])

    def start_g(j, b):
        pltpu.async_copy(
            hp_hbm.at[srcl_v.at[pl.ds(j * K3K, K3K)]], rows[b], gsems[b]
        )

    def wait_g(j, b):
        pltpu.make_async_copy(
            hp_hbm.at[srcl_v.at[pl.ds(j * K3K, K3K)]], rows[b], gsems[b]
        ).wait()

    def start_s(j, b):
        pltpu.async_copy(rows[b], acc_sh.at[dstl_v.at[j]], ssems[b], add=True)

    def wait_s(j, b):
        pltpu.make_async_copy(rows[b], acc_sh.at[dstl_v.at[j]], ssems[b]).wait()

    def edge_pass():
        start_g(0, 0)
        start_g(1, 1)

        def gbody(g, carry):
            for b in range(4):
                j = g * 4 + b
                wait_g(j, b)
                start_s(j, b)
                bn = (b + 2) % 4

                @pl.when(j >= 2)
                def _():
                    wait_s(j - 2, bn)

                @pl.when(j + 2 < K3CH)
                def _():
                    start_g(j + 2, bn)
            return carry

        lax.fori_loop(0, K3CH // 4, gbody, 0)

        jl = K3CH - 1
        wait_g(jl, 0)
        start_s(jl, 0)
        wait_s(jl - 2, 2)
        wait_s(jl - 1, 3)
        wait_s(jl, 0)

    for p in range(2):
        zero_acc()
        shift_src(c * N if p == 0 else 2 * N)
        plsc.subcore_barrier()
        edge_pass()
        plsc.subcore_barrier()
        pltpu.sync_copy(
            acc_sh.at[pl.ds(s * SEG, SEG)],
            out_hbm.at[2 * p + c, pl.ds(s * SEG, SEG)],
        )


def _k3_call(hp2, src, dst3):
    kfn = pl.kernel(
        _k3_body,
        out_type=jax.ShapeDtypeStruct((NQ, NPAD, DQ), jnp.float32),
        mesh=_sc_mesh(),
        compiler_params=pltpu.CompilerParams(use_tc_tiling_on_sc=False),
        scratch_types=[
            pltpu.VMEM((E // NS,), jnp.int32),
            pltpu.VMEM((K3CH, K3K), jnp.int32),
            pltpu.VMEM((K3K, DQ), jnp.float32),
            pltpu.VMEM((K3K, DQ), jnp.float32),
            pltpu.VMEM((K3K, DQ), jnp.float32),
            pltpu.VMEM((K3K, DQ), jnp.float32),
            pltpu.SemaphoreType.DMA,
            pltpu.SemaphoreType.DMA,
            pltpu.SemaphoreType.DMA,
            pltpu.SemaphoreType.DMA,
            pltpu.SemaphoreType.DMA,
            pltpu.SemaphoreType.DMA,
            pltpu.SemaphoreType.DMA,
            pltpu.SemaphoreType.DMA,
            pltpu.VMEM_SHARED((NPAD, DQ), jnp.float32),
        ],
    )
    return kfn(hp2, src, dst3)


def _d1_body(acc_ref, hp_ref, p_ref, t_ref, stats_ref):
    i = pl.program_id(0)
    p = p_ref[...]
    deg = p[0] + p[1] + 1.0
    dinv = lax.rsqrt(deg[:, 0:1])
    a = acc_ref[...]
    hp = hp_ref[...]
    tb = jnp.concatenate(
        [a[q] + hp[q] for q in range(NQ)], axis=1
    ) * dinv
    t_ref[...] = tb

    @pl.when(i == 0)
    def _():
        stats_ref[...] = jnp.zeros_like(stats_ref)

    stats_ref[...] += jnp.stack([jnp.sum(tb, 0), jnp.sum(tb * tb, 0)], 0)


def _d1_call(acc, hp, partials):
    return pl.pallas_call(
        _d1_body,
        grid=(NBLK,),
        in_specs=[
            pl.BlockSpec((NQ, RBLK, DQ), lambda i: (0, i, 0)),
            pl.BlockSpec((NQ, RBLK, DQ), lambda i: (0, i, 0)),
            pl.BlockSpec((NC, RBLK, 16), lambda i: (0, i, 0)),
        ],
        out_specs=[
            pl.BlockSpec((RBLK, D), lambda i: (i, 0)),
            pl.BlockSpec((2, D), lambda i: (0, 0)),
        ],
        out_shape=[
            jax.ShapeDtypeStruct((N, D), jnp.float32),
            jax.ShapeDtypeStruct((2, D), jnp.float32),
        ],
    )(acc, hp, partials)


def _d2_body(t_ref, stats_ref, g_ref, b_ref, y_ref):
    st = stats_ref[...]
    mean = st[0:1] * (1.0 / N)
    ex2 = st[1:2] * (1.0 / N)
    var = ex2 - mean * mean
    scale = lax.rsqrt(var + BN_EPS_) * g_ref[...]
    y = (t_ref[...] - mean) * scale + b_ref[...]
    y_ref[...] = jnp.maximum(y, 0.0)


def _d2_call(t, stats, gamma, beta):
    return pl.pallas_call(
        _d2_body,
        grid=(NBLK,),
        in_specs=[
            pl.BlockSpec((RBLK, D), lambda i: (i, 0)),
            pl.BlockSpec((2, D), lambda i: (0, 0)),
            pl.BlockSpec((1, D), lambda i: (0, 0)),
            pl.BlockSpec((1, D), lambda i: (0, 0)),
        ],
        out_specs=pl.BlockSpec((RBLK, D), lambda i: (i, 0)),
        out_shape=jax.ShapeDtypeStruct((N, D), jnp.float32),
    )(t, stats, gamma, beta)


def kernel(x, edge_index, W, bn_gamma, bn_beta):
    ei = edge_index.astype(jnp.int32)
    src = ei[0]
    dst = ei[1]
    dst_k1 = dst.reshape(NC * NS, K1CH, K1K)
    dst_k3 = dst.reshape(NS, K3CH, K3K)

    partials = _k1_call(dst_k1)
    hp = _k2_call(x, W, partials)
    acc = _k3_call(hp.reshape(NQ * N, DQ), src, dst_k3)
    t, stats = _d1_call(acc, hp, partials)
    return _d2_call(t, stats, bn_gamma.reshape(1, D), bn_beta.reshape(1, D))


# speedup vs baseline: 18.8805x; 1.3205x over previous
import jax
import jax.numpy as jnp
from jax import lax
from jax.experimental import pallas as pl
from jax.experimental.pallas import tpu as pltpu
from jax.experimental.pallas import tpu_sc as plsc

N = 10000
E = 160000
D = 256
DH = 128
NPAD = 10240
NC = 2
NS = 16
SEG = NPAD // NS

K1K = 40
K1CH = (E // (NC * NS)) // K1K

K3K = 80
K3CH = (E // NS) // K3K

BN_EPS_ = 1e-5
RBLK = 400
NBLK = N // RBLK


def _sc_mesh():
    return plsc.VectorSubcoreMesh(
        core_axis_name="c", subcore_axis_name="s", num_cores=NC, num_subcores=NS
    )


def _k1_body(dst_hbm, out_hbm, ones_v, zb_v, dstc_v, deg_sh):
    c = lax.axis_index("c")
    s = lax.axis_index("s")
    w = c * NS + s

    def fill(i, carry):
        zb_v[i] = jnp.zeros((16,), jnp.float32)
        ones_v[i] = jnp.ones((16,), jnp.float32)
        return carry

    lax.fori_loop(0, K1K, fill, 0)

    for r in range(SEG // K1K):
        pltpu.sync_copy(zb_v, deg_sh.at[pl.ds(s * SEG + r * K1K, K1K)])
    plsc.subcore_barrier()

    pltpu.sync_copy(dst_hbm.at[w], dstc_v)

    def body(j, carry):
        pltpu.sync_copy(ones_v, deg_sh.at[dstc_v.at[j]], add=True)
        return carry

    lax.fori_loop(0, K1CH, body, 0)
    plsc.subcore_barrier()
    pltpu.sync_copy(
        deg_sh.at[pl.ds(s * SEG, SEG)], out_hbm.at[c, pl.ds(s * SEG, SEG)]
    )


def _k1_call(dst3):
    kfn = pl.kernel(
        _k1_body,
        out_type=jax.ShapeDtypeStruct((NC, NPAD, 16), jnp.float32),
        mesh=_sc_mesh(),
        compiler_params=pltpu.CompilerParams(use_tc_tiling_on_sc=False),
        scratch_types=[
            pltpu.VMEM((K1K, 16), jnp.float32),
            pltpu.VMEM((K1K, 16), jnp.float32),
            pltpu.VMEM((K1CH, K1K), jnp.int32),
            pltpu.VMEM_SHARED((NPAD, 16), jnp.float32),
        ],
    )
    return kfn(dst3)


def _k2_body(x_ref, w_ref, p_ref, hp_ref):
    h = jnp.dot(x_ref[...], w_ref[...], preferred_element_type=jnp.float32)
    p = p_ref[...]
    deg = p[0] + p[1] + 1.0
    dinv = lax.rsqrt(deg[:, 0:1])
    hp_ref[0] = (h[:, :DH] * dinv).astype(jnp.bfloat16)
    hp_ref[1] = (h[:, DH:] * dinv).astype(jnp.bfloat16)


def _k2_call(x, W, partials):
    return pl.pallas_call(
        _k2_body,
        grid=(NBLK,),
        in_specs=[
            pl.BlockSpec((RBLK, D), lambda i: (i, 0)),
            pl.BlockSpec((D, D), lambda i: (0, 0)),
            pl.BlockSpec((NC, RBLK, 16), lambda i: (0, i, 0)),
        ],
        out_specs=pl.BlockSpec((NC, RBLK, DH), lambda i: (0, i, 0)),
        out_shape=jax.ShapeDtypeStruct((NC, N, DH), jnp.bfloat16),
    )(x, W, partials)


def _k3_body(hp_hbm, src_hbm, dst_hbm, out_hbm,
             srcl_v, dstl_v, rows0, rows1, rows2, rows3,
             gsem0, gsem1, gsem2, gsem3, ssem0, ssem1, ssem2, ssem3,
             acc_sh):
    c = lax.axis_index("c")
    s = lax.axis_index("s")
    epp = E // NS

    rows = (rows0, rows1, rows2, rows3)
    gsems = (gsem0, gsem1, gsem2, gsem3)
    ssems = (ssem0, ssem1, ssem2, ssem3)

    def zf(i, carry):
        for q in range(DH // 32):
            rows0[i, pl.ds(q * 32, 32)] = jnp.zeros((32,), jnp.bfloat16)
        return carry

    lax.fori_loop(0, K3K, zf, 0)
    for r in range(SEG // K3K):
        pltpu.sync_copy(rows0, acc_sh.at[pl.ds(s * SEG + r * K3K, K3K)])

    pltpu.sync_copy(src_hbm.at[pl.ds(s * epp, epp)], srcl_v)
    pltpu.sync_copy(dst_hbm.at[s], dstl_v)
    off = c * N

    def adj(j, carry):
        srcl_v[pl.ds(j * 16, 16)] = srcl_v[pl.ds(j * 16, 16)] + off
        return carry

    lax.fori_loop(0, epp // 16, adj, 0)
    plsc.subcore_barrier()

    def start_g(j, b):
        pltpu.async_copy(
            hp_hbm.at[srcl_v.at[pl.ds(j * K3K, K3K)]], rows[b], gsems[b]
        )

    def wait_g(j, b):
        pltpu.make_async_copy(
            hp_hbm.at[srcl_v.at[pl.ds(j * K3K, K3K)]], rows[b], gsems[b]
        ).wait()

    def start_s(j, b):
        pltpu.async_copy(rows[b], acc_sh.at[dstl_v.at[j]], ssems[b], add=True)

    def wait_s(j, b):
        pltpu.make_async_copy(rows[b], acc_sh.at[dstl_v.at[j]], ssems[b]).wait()

    start_g(0, 0)
    start_g(1, 1)

    def gbody(g, carry):
        for b in range(4):
            j = g * 4 + b
            wait_g(j, b)
            start_s(j, b)
            bn = (b + 2) % 4

            @pl.when(j >= 2)
            def _():
                wait_s(j - 2, bn)

            @pl.when(j + 2 < K3CH)
            def _():
                start_g(j + 2, bn)
        return carry

    lax.fori_loop(0, K3CH // 4, gbody, 0)

    jl = K3CH - 1
    wait_g(jl, 0)
    start_s(jl, 0)
    wait_s(jl - 2, 2)
    wait_s(jl - 1, 3)
    wait_s(jl, 0)

    plsc.subcore_barrier()
    pltpu.sync_copy(
        acc_sh.at[pl.ds(s * SEG, SEG)], out_hbm.at[c, pl.ds(s * SEG, SEG)]
    )


def _k3_call(hp2, src, dst3):
    kfn = pl.kernel(
        _k3_body,
        out_type=jax.ShapeDtypeStruct((NC, NPAD, DH), jnp.bfloat16),
        mesh=_sc_mesh(),
        compiler_params=pltpu.CompilerParams(use_tc_tiling_on_sc=False),
        scratch_types=[
            pltpu.VMEM((E // NS,), jnp.int32),
            pltpu.VMEM((K3CH, K3K), jnp.int32),
            pltpu.VMEM((K3K, DH), jnp.bfloat16),
            pltpu.VMEM((K3K, DH), jnp.bfloat16),
            pltpu.VMEM((K3K, DH), jnp.bfloat16),
            pltpu.VMEM((K3K, DH), jnp.bfloat16),
            pltpu.SemaphoreType.DMA,
            pltpu.SemaphoreType.DMA,
            pltpu.SemaphoreType.DMA,
            pltpu.SemaphoreType.DMA,
            pltpu.SemaphoreType.DMA,
            pltpu.SemaphoreType.DMA,
            pltpu.SemaphoreType.DMA,
            pltpu.SemaphoreType.DMA,
            pltpu.VMEM_SHARED((NPAD, DH), jnp.bfloat16),
        ],
    )
    return kfn(hp2, src, dst3)


def _dinv_of(p):
    deg = p[0] + p[1] + 1.0
    return lax.rsqrt(deg[:, 0:1])


def _t_block(acc_ref, hp_ref, p_ref):
    dinv = _dinv_of(p_ref[...])
    a = acc_ref[...].astype(jnp.float32)
    hp = hp_ref[...].astype(jnp.float32)
    return jnp.concatenate([a[0] + hp[0], a[1] + hp[1]], axis=1) * dinv


def _d1_body(acc_ref, hp_ref, p_ref, stats_ref):
    i = pl.program_id(0)
    tb = _t_block(acc_ref, hp_ref, p_ref)

    @pl.when(i == 0)
    def _():
        stats_ref[...] = jnp.zeros_like(stats_ref)

    stats_ref[...] += jnp.stack([jnp.sum(tb, 0), jnp.sum(tb * tb, 0)], 0)


def _d1_call(acc, hp, partials):
    return pl.pallas_call(
        _d1_body,
        grid=(NBLK,),
        in_specs=[
            pl.BlockSpec((NC, RBLK, DH), lambda i: (0, i, 0)),
            pl.BlockSpec((NC, RBLK, DH), lambda i: (0, i, 0)),
            pl.BlockSpec((NC, RBLK, 16), lambda i: (0, i, 0)),
        ],
        out_specs=pl.BlockSpec((2, D), lambda i: (0, 0)),
        out_shape=jax.ShapeDtypeStruct((2, D), jnp.float32),
    )(acc, hp, partials)


def _d2_body(acc_ref, hp_ref, p_ref, stats_ref, g_ref, b_ref, y_ref):
    tb = _t_block(acc_ref, hp_ref, p_ref)
    st = stats_ref[...]
    mean = st[0:1] * (1.0 / N)
    ex2 = st[1:2] * (1.0 / N)
    var = ex2 - mean * mean
    scale = lax.rsqrt(var + BN_EPS_) * g_ref[...]
    y = (tb - mean) * scale + b_ref[...]
    y_ref[...] = jnp.maximum(y, 0.0)


def _d2_call(acc, hp, partials, stats, gamma, beta):
    return pl.pallas_call(
        _d2_body,
        grid=(NBLK,),
        in_specs=[
            pl.BlockSpec((NC, RBLK, DH), lambda i: (0, i, 0)),
            pl.BlockSpec((NC, RBLK, DH), lambda i: (0, i, 0)),
            pl.BlockSpec((NC, RBLK, 16), lambda i: (0, i, 0)),
            pl.BlockSpec((2, D), lambda i: (0, 0)),
            pl.BlockSpec((1, D), lambda i: (0, 0)),
            pl.BlockSpec((1, D), lambda i: (0, 0)),
        ],
        out_specs=pl.BlockSpec((RBLK, D), lambda i: (i, 0)),
        out_shape=jax.ShapeDtypeStruct((N, D), jnp.float32),
    )(acc, hp, partials, stats, gamma, beta)


def kernel(x, edge_index, W, bn_gamma, bn_beta):
    ei = edge_index.astype(jnp.int32)
    src = ei[0]
    dst = ei[1]
    dst_k1 = dst.reshape(NC * NS, K1CH, K1K)
    dst_k3 = dst.reshape(NS, K3CH, K3K)

    partials = _k1_call(dst_k1)
    hp = _k2_call(x, W, partials)
    acc = _k3_call(hp.reshape(NC * N, DH), src, dst_k3)
    stats = _d1_call(acc, hp, partials)
    return _d2_call(acc, hp, partials, stats,
                    bn_gamma.reshape(1, D), bn_beta.reshape(1, D))
